# baseline (speedup 1.0000x reference)
import functools

import jax
import jax.numpy as jnp
from jax import lax
from jax.experimental import pallas as pl
from jax.experimental.pallas import tpu as pltpu
from jax.experimental.pallas import tpu_sc as plsc

KCB = 8192
DIM = 32
NTOK = 4 * 16 * 16
KTILE = 512
NKT = KCB // KTILE
COMMIT = 0.25


def _argmin_body(xT_ref, x2_ref, e2_ref, E_ref, idx_ref, d2sum_ref,
                 bestm_ref, bestid_ref):
    j = pl.program_id(0)
    cross = lax.dot_general(
        E_ref[...], xT_ref[...], (((1,), (0,)), ((), ())),
        preferred_element_type=jnp.float32)
    d2 = jnp.maximum(x2_ref[...] - 2.0 * cross + e2_ref[...], 0.0)
    dist = jnp.sqrt(d2)
    m = jnp.min(dist, axis=0, keepdims=True)
    kidx = j * KTILE + lax.broadcasted_iota(jnp.int32, (KTILE, NTOK), 0)
    lid = jnp.min(jnp.where(dist == m, kidx, KCB), axis=0, keepdims=True)

    @pl.when(j == 0)
    def _():
        bestm_ref[...] = m
        bestid_ref[...] = lid

    @pl.when(j > 0)
    def _():
        bm = bestm_ref[...]
        better = m < bm
        bestm_ref[...] = jnp.where(better, m, bm)
        bestid_ref[...] = jnp.where(better, lid, bestid_ref[...])

    @pl.when(j == NKT - 1)
    def _():
        idx_ref[...] = bestid_ref[...]
        bm = bestm_ref[...]
        d2sum_ref[...] = jnp.sum(bm * bm, keepdims=True).reshape(1, 1)


_argmin_call = pl.pallas_call(
    _argmin_body,
    grid=(NKT,),
    in_specs=[
        pl.BlockSpec((DIM, NTOK), lambda j: (0, 0)),
        pl.BlockSpec((1, NTOK), lambda j: (0, 0)),
        pl.BlockSpec((KTILE, 1), lambda j: (j, 0)),
        pl.BlockSpec((KTILE, DIM), lambda j: (j, 0)),
    ],
    out_specs=[
        pl.BlockSpec((1, NTOK), lambda j: (0, 0)),
        pl.BlockSpec((1, 1), lambda j: (0, 0)),
    ],
    out_shape=[
        jax.ShapeDtypeStruct((1, NTOK), jnp.int32),
        jax.ShapeDtypeStruct((1, 1), jnp.float32),
    ],
    scratch_shapes=[
        pltpu.VMEM((1, NTOK), jnp.float32),
        pltpu.VMEM((1, NTOK), jnp.int32),
    ],
)

_NC, _NS = 2, 16
_NW = _NC * _NS
_BPW = NTOK // _NW


_GROW = 128
_NGR = KCB * DIM // _GROW


@functools.cache
def _sc_gather_fn():
    mesh = plsc.VectorSubcoreMesh(core_axis_name="c", subcore_axis_name="s")

    @functools.partial(
        pl.kernel,
        mesh=mesh,
        out_type=jax.ShapeDtypeStruct((NTOK, _GROW), jnp.float32),
        scratch_types=[
            pltpu.VMEM((_BPW,), jnp.int32),
            pltpu.VMEM((_BPW, _GROW), jnp.float32),
            pltpu.SemaphoreType.DMA,
        ],
    )
    def _sc_gather(table_hbm, idx_hbm, out_hbm, idx_v, rows_v, sem):
        wid = lax.axis_index("s") * _NC + lax.axis_index("c")
        base = wid * _BPW
        pltpu.sync_copy(idx_hbm.at[pl.ds(base, _BPW)], idx_v)
        pltpu.async_copy(table_hbm.at[idx_v], rows_v, sem).wait()
        pltpu.sync_copy(rows_v, out_hbm.at[pl.ds(base, _BPW)])

    return _sc_gather


def _select_body(g_ref, off_ref, zq_ref):
    off = off_ref[...]
    acc = jnp.zeros((NTOK, DIM), jnp.float32)
    for j in range(_GROW // DIM):
        acc = jnp.where(off == j, g_ref[:, j * DIM:(j + 1) * DIM], acc)
    zq_ref[...] = acc


_select_call = pl.pallas_call(
    _select_body,
    in_specs=[
        pl.BlockSpec((NTOK, _GROW), lambda: (0, 0)),
        pl.BlockSpec((NTOK, 1), lambda: (0, 0)),
    ],
    out_specs=pl.BlockSpec((NTOK, DIM), lambda: (0, 0)),
    out_shape=jax.ShapeDtypeStruct((NTOK, DIM), jnp.float32),
)


def kernel(x, embedding_weight):
    b, d, h, w = x.shape
    xp = jnp.moveaxis(x, 1, -1)
    x2 = jnp.sum(xp * xp, axis=-1, keepdims=True)
    e2 = jnp.sum(embedding_weight * embedding_weight, axis=-1)

    xT = xp.reshape(NTOK, DIM).T
    idx2d, d2sum = _argmin_call(xT, x2.reshape(1, NTOK),
                                e2.reshape(KCB, 1), embedding_weight)

    idx = idx2d.reshape(NTOK)
    blocks = _sc_gather_fn()(
        embedding_weight.reshape(_NGR, _GROW), idx >> 2)
    rows = _select_call(blocks, (idx & 3).reshape(NTOK, 1))
    zq = jnp.moveaxis(rows.reshape(b, h, w, d), -1, 1)

    quantized = x + (zq - x)
    mse = d2sum[0, 0] / (NTOK * DIM)
    vq_loss = mse + COMMIT * mse
    return quantized, vq_loss


# speedup vs baseline: 1.0535x; 1.0535x over previous
import functools

import jax
import jax.numpy as jnp
from jax import lax
from jax.experimental import pallas as pl
from jax.experimental.pallas import tpu as pltpu
from jax.experimental.pallas import tpu_sc as plsc

KCB = 8192
DIM = 32
NTOK = 4 * 16 * 16
KTILE = 512
NKT = KCB // KTILE
COMMIT = 0.25


def _argmin_body(xT_ref, x2_ref, e2_ref, E_ref, idx_ref, d2sum_ref,
                 bestm_ref, bestid_ref):
    j = pl.program_id(0)
    cross2 = lax.dot_general(
        E_ref[...], 2.0 * xT_ref[...], (((1,), (0,)), ((), ())),
        preferred_element_type=jnp.float32)
    dist = jnp.sqrt((x2_ref[...] - cross2) + e2_ref[...])
    m = jnp.min(dist, axis=0, keepdims=True)
    kidx = lax.broadcasted_iota(jnp.int32, (KTILE, NTOK), 0).astype(jnp.float32)
    lid = jnp.min(jnp.where(dist <= m, kidx, float(KTILE)),
                  axis=0, keepdims=True)

    @pl.when(j == 0)
    def _():
        bestm_ref[...] = m
        bestid_ref[...] = lid

    @pl.when(j > 0)
    def _():
        bm = bestm_ref[...]
        better = m < bm
        bestm_ref[...] = jnp.where(better, m, bm)
        bestid_ref[...] = jnp.where(better, lid + (j * KTILE), bestid_ref[...])

    @pl.when(j == NKT - 1)
    def _():
        idx_ref[...] = bestid_ref[...].astype(jnp.int32)
        bm = bestm_ref[...]
        d2sum_ref[...] = jnp.sum(bm * bm, keepdims=True).reshape(1, 1)


_argmin_call = pl.pallas_call(
    _argmin_body,
    grid=(NKT,),
    in_specs=[
        pl.BlockSpec((DIM, NTOK), lambda j: (0, 0)),
        pl.BlockSpec((1, NTOK), lambda j: (0, 0)),
        pl.BlockSpec((KTILE, 1), lambda j: (j, 0)),
        pl.BlockSpec((KTILE, DIM), lambda j: (j, 0)),
    ],
    out_specs=[
        pl.BlockSpec((1, NTOK), lambda j: (0, 0)),
        pl.BlockSpec((1, 1), lambda j: (0, 0)),
    ],
    out_shape=[
        jax.ShapeDtypeStruct((1, NTOK), jnp.int32),
        jax.ShapeDtypeStruct((1, 1), jnp.float32),
    ],
    scratch_shapes=[
        pltpu.VMEM((1, NTOK), jnp.float32),
        pltpu.VMEM((1, NTOK), jnp.float32),
    ],
)

_NC, _NS = 2, 16
_NW = _NC * _NS
_BPW = NTOK // _NW


_GROW = 128
_NGR = KCB * DIM // _GROW


@functools.cache
def _sc_gather_fn():
    mesh = plsc.VectorSubcoreMesh(core_axis_name="c", subcore_axis_name="s")

    @functools.partial(
        pl.kernel,
        mesh=mesh,
        out_type=jax.ShapeDtypeStruct((NTOK, _GROW), jnp.float32),
        scratch_types=[
            pltpu.VMEM((_BPW,), jnp.int32),
            pltpu.VMEM((_BPW, _GROW), jnp.float32),
            pltpu.SemaphoreType.DMA,
        ],
    )
    def _sc_gather(table_hbm, idx_hbm, out_hbm, idx_v, rows_v, sem):
        wid = lax.axis_index("s") * _NC + lax.axis_index("c")
        base = wid * _BPW
        pltpu.sync_copy(idx_hbm.at[pl.ds(base, _BPW)], idx_v)
        pltpu.async_copy(table_hbm.at[idx_v], rows_v, sem).wait()
        pltpu.sync_copy(rows_v, out_hbm.at[pl.ds(base, _BPW)])

    return _sc_gather


def _select_body(g_ref, off_ref, zq_ref):
    off = off_ref[...]
    acc = jnp.zeros((NTOK, DIM), jnp.float32)
    for j in range(_GROW // DIM):
        acc = jnp.where(off == j, g_ref[:, j * DIM:(j + 1) * DIM], acc)
    zq_ref[...] = acc


_select_call = pl.pallas_call(
    _select_body,
    in_specs=[
        pl.BlockSpec((NTOK, _GROW), lambda: (0, 0)),
        pl.BlockSpec((NTOK, 1), lambda: (0, 0)),
    ],
    out_specs=pl.BlockSpec((NTOK, DIM), lambda: (0, 0)),
    out_shape=jax.ShapeDtypeStruct((NTOK, DIM), jnp.float32),
)


def kernel(x, embedding_weight):
    b, d, h, w = x.shape
    xp = jnp.moveaxis(x, 1, -1)
    x2 = jnp.sum(xp * xp, axis=-1, keepdims=True)
    e2 = jnp.sum(embedding_weight * embedding_weight, axis=-1)

    xT = xp.reshape(NTOK, DIM).T
    idx2d, d2sum = _argmin_call(xT, x2.reshape(1, NTOK),
                                e2.reshape(KCB, 1), embedding_weight)

    idx = idx2d.reshape(NTOK)
    blocks = _sc_gather_fn()(
        embedding_weight.reshape(_NGR, _GROW), idx >> 2)
    rows = _select_call(blocks, (idx & 3).reshape(NTOK, 1))
    zq = jnp.moveaxis(rows.reshape(b, h, w, d), -1, 1)

    quantized = x + (zq - x)
    mse = d2sum[0, 0] / (NTOK * DIM)
    vq_loss = mse + COMMIT * mse
    return quantized, vq_loss
